# baseline (device time: 223659 ns/iter reference)
import jax
import jax.numpy as jnp
from jax import lax
from jax.experimental import pallas as pl
from jax.experimental.pallas import tpu as pltpu

N_DEV = 16
B, SQ, DMODEL = 2, 512, 768
H, DH = 8, 64
HD = H * DH
BLK = 64


def _allreduce_body(num_ref, l_ref, out_num_ref, out_l_ref,
                    comm_num, comm_l, send_n, recv_n, send_l, recv_l):
    my = lax.axis_index("i")
    left = (my - 1) % N_DEV
    right = (my + 1) % N_DEV

    barrier = pltpu.get_barrier_semaphore()
    for nbr in (left, right):
        pl.semaphore_signal(barrier, inc=1, device_id=(nbr,),
                            device_id_type=pl.DeviceIdType.MESH)
    pl.semaphore_wait(barrier, 2)

    comm_num[0] = num_ref[...]
    comm_l[0] = l_ref[...]
    out_num_ref[...] = num_ref[...].astype(jnp.float32)
    out_l_ref[...] = l_ref[...]

    for h in range(N_DEV - 1):
        rn = pltpu.make_async_remote_copy(
            src_ref=comm_num.at[h], dst_ref=comm_num.at[h + 1],
            send_sem=send_n.at[h], recv_sem=recv_n.at[h],
            device_id=(right,), device_id_type=pl.DeviceIdType.MESH)
        rl = pltpu.make_async_remote_copy(
            src_ref=comm_l.at[h], dst_ref=comm_l.at[h + 1],
            send_sem=send_l.at[h], recv_sem=recv_l.at[h],
            device_id=(right,), device_id_type=pl.DeviceIdType.MESH)
        rn.start()
        rl.start()
        rn.wait()
        rl.wait()
        out_num_ref[...] += comm_num[h + 1].astype(jnp.float32)
        out_l_ref[...] += comm_l[h + 1]


def _allreduce(num, l):
    return pl.pallas_call(
        _allreduce_body,
        out_shape=[
            jax.ShapeDtypeStruct((B, SQ, HD), jnp.float32),
            jax.ShapeDtypeStruct((B, H, SQ), jnp.float32),
        ],
        in_specs=[pl.BlockSpec(memory_space=pltpu.VMEM),
                  pl.BlockSpec(memory_space=pltpu.VMEM)],
        out_specs=[pl.BlockSpec(memory_space=pltpu.VMEM),
                   pl.BlockSpec(memory_space=pltpu.VMEM)],
        scratch_shapes=[
            pltpu.VMEM((N_DEV, B, SQ, HD), jnp.bfloat16),
            pltpu.VMEM((N_DEV, B, H, SQ), jnp.float32),
            pltpu.SemaphoreType.DMA((N_DEV - 1,)),
            pltpu.SemaphoreType.DMA((N_DEV - 1,)),
            pltpu.SemaphoreType.DMA((N_DEV - 1,)),
            pltpu.SemaphoreType.DMA((N_DEV - 1,)),
        ],
        compiler_params=pltpu.CompilerParams(collective_id=0),
    )(num, l)


def kernel(x, Wq, K_ext, V_ext, Wo):
    bf = jnp.bfloat16
    Q = (x.astype(bf) @ Wq.astype(bf)).reshape(B, SQ, H, DH)
    K = K_ext.astype(bf)
    V = V_ext.astype(bf)

    blk = jnp.arange(SQ) // BLK
    mask = (blk[:, None] % 4) == (blk[None, :] % 4)

    s = jnp.einsum("bihd,bjhd->bhij", Q, K,
                   preferred_element_type=jnp.float32) * 0.125
    w = jnp.where(mask[None, None], jnp.exp(s), 0.0)
    l = jnp.sum(w, axis=-1)
    num = jnp.einsum("bhij,bjhd->bihd", w.astype(bf), V,
                     preferred_element_type=jnp.float32)

    num_sum, l_sum = _allreduce(num.reshape(B, SQ, HD).astype(bf), l)

    ctx = num_sum.reshape(B, SQ, H, DH) / l_sum.transpose(0, 2, 1)[..., None]
    out = ctx.reshape(B, SQ, HD).astype(bf) @ Wo.astype(bf)
    return out.astype(jnp.float32)


# device time: 63453 ns/iter; 3.5248x vs baseline; 3.5248x over previous
import jax
import jax.numpy as jnp
from jax import lax
from jax.experimental import pallas as pl
from jax.experimental.pallas import tpu as pltpu

N_DEV = 16
B, SQ, DMODEL = 2, 512, 768
H, DH = 8, 64
HD = H * DH
BLK = 64
LSTRIDE = 8
ROWS = 2 * SQ + (SQ // 32) * LSTRIDE

_MESH = pl.DeviceIdType.MESH


def _allreduce_body(in_ref, out_ref, stag, send_sems, recv_sems):
    my = lax.axis_index("i")

    barrier = pltpu.get_barrier_semaphore()
    for k in range(4):
        pl.semaphore_signal(barrier, inc=1, device_id=(my ^ (1 << k),),
                            device_id_type=_MESH)
    pl.semaphore_wait(barrier, 4)

    out_ref[...] = in_ref[...]

    off = 0
    ln = SQ
    for r in range(4):
        half = ln // 2
        bit = (my >> r) & 1
        send_off = off + (1 - bit) * half
        keep_off = off + bit * half
        partner = my ^ (1 << r)
        lh = (half // 32) * LSTRIDE
        copies = []
        for p, (so, do, n) in enumerate([
                (send_off, 0, half),
                (SQ + send_off, half, half),
                (2 * SQ + (send_off // 32) * LSTRIDE, 2 * half, lh)]):
            copies.append(pltpu.make_async_remote_copy(
                src_ref=out_ref.at[pl.ds(so, n)],
                dst_ref=stag.at[r, pl.ds(do, n)],
                send_sem=send_sems.at[3 * r + p],
                recv_sem=recv_sems.at[3 * r + p],
                device_id=(partner,), device_id_type=_MESH))
        for c in copies:
            c.start()
        for c in copies:
            c.wait()
        for ko, do, n in [(keep_off, 0, half),
                          (SQ + keep_off, half, half),
                          (2 * SQ + (keep_off // 32) * LSTRIDE, 2 * half, lh)]:
            rows = pl.ds(ko, n)
            out_ref[rows] = (
                out_ref[rows].astype(jnp.float32)
                + stag[r, pl.ds(do, n)].astype(jnp.float32)
            ).astype(jnp.bfloat16)
        off = keep_off
        ln = half

    s = off
    seg = ln
    for i, k in enumerate((3, 2, 1, 0)):
        partner = my ^ (1 << k)
        base = 3 * (4 + i)
        lh = (seg // 32) * LSTRIDE
        copies = []
        for p, (so, n) in enumerate([(s, seg), (SQ + s, seg),
                                     (2 * SQ + (s // 32) * LSTRIDE, lh)]):
            copies.append(pltpu.make_async_remote_copy(
                src_ref=out_ref.at[pl.ds(so, n)],
                dst_ref=out_ref.at[pl.ds(so, n)],
                send_sem=send_sems.at[base + p],
                recv_sem=recv_sems.at[base + p],
                device_id=(partner,), device_id_type=_MESH))
        for c in copies:
            c.start()
        for c in copies:
            c.wait()
        s = s - ((my >> k) & 1) * seg
        seg = seg * 2


def _allreduce(packed):
    return pl.pallas_call(
        _allreduce_body,
        out_shape=jax.ShapeDtypeStruct((ROWS, HD), jnp.bfloat16),
        in_specs=[pl.BlockSpec(memory_space=pltpu.VMEM)],
        out_specs=pl.BlockSpec(memory_space=pltpu.VMEM),
        scratch_shapes=[
            pltpu.VMEM((4, SQ + (SQ // 64) * LSTRIDE, HD), jnp.bfloat16),
            pltpu.SemaphoreType.DMA((24,)),
            pltpu.SemaphoreType.DMA((24,)),
        ],
        compiler_params=pltpu.CompilerParams(collective_id=0),
    )(packed)


def kernel(x, Wq, K_ext, V_ext, Wo):
    bf = jnp.bfloat16
    Q = (x.astype(bf) @ Wq.astype(bf)).reshape(B, SQ, H, DH)
    K = K_ext.astype(bf)
    V = V_ext.astype(bf)

    blk = jnp.arange(SQ) // BLK
    mask = (blk[:, None] % 4) == (blk[None, :] % 4)

    s = jnp.einsum("bihd,bjhd->bhij", Q, K,
                   preferred_element_type=jnp.float32) * 0.125
    w = jnp.where(mask[None, None], jnp.exp(s), 0.0)
    l = jnp.sum(w, axis=-1)
    num = jnp.einsum("bhij,bjhd->bihd", w.astype(bf), V,
                     preferred_element_type=jnp.float32)

    num2 = num.reshape(B * SQ, HD).astype(bf)
    l_pack = l.transpose(2, 0, 1).reshape(SQ // 32, 1, HD).astype(bf)
    l_pack = jnp.pad(l_pack, ((0, 0), (0, LSTRIDE - 1), (0, 0)))
    packed = jnp.concatenate([num2, l_pack.reshape(-1, HD)], axis=0)

    out = _allreduce(packed)
    num_sum = out[: 2 * SQ].astype(jnp.float32).reshape(B, SQ, H, DH)
    l_sum = out[2 * SQ :].reshape(SQ // 32, LSTRIDE, HD)[:, 0].reshape(SQ, B, H).astype(jnp.float32)

    ctx = num_sum / l_sum.transpose(1, 0, 2)[..., None]
    y = ctx.reshape(B, SQ, HD).astype(bf) @ Wo.astype(bf)
    return y.astype(jnp.float32)
